# baseline (speedup 1.0000x reference)
import functools

import jax
import jax.numpy as jnp
from jax import lax
from jax.experimental import pallas as pl
from jax.experimental.pallas import tpu as pltpu
from jax.experimental.pallas import tpu_sc as plsc

SEQ_LEN = 8192
EMBED_DIM = 1024

_info = plsc.get_sparse_core_info()
_NC, _NS = _info.num_cores, _info.num_subcores
_NW = _NC * _NS
_ROWS_PER_W = SEQ_LEN // _NW
_CH = 32
_NCHUNK = _ROWS_PER_W // _CH

_mesh = plsc.VectorSubcoreMesh(core_axis_name="c", subcore_axis_name="s")


def _pipeline_ops(n, nbuf=2):
    ops = [("si", i) for i in range(nbuf)]
    ops += [("wi", 0), ("so", 0)]
    for i in range(1, n):
        ops += [("wi", i), ("so", i), ("wo", i - 1)]
        if i + 1 < n:
            ops.append(("si", i + 1))
    ops.append(("wo", n - 1))
    return ops


@functools.partial(
    pl.kernel,
    mesh=_mesh,
    out_type=jax.ShapeDtypeStruct((SEQ_LEN, EMBED_DIM), jnp.float32),
    scratch_types=[
        pltpu.VMEM((2, _CH, EMBED_DIM), jnp.float32),
        pltpu.VMEM_SHARED((_NS, 2, _CH, EMBED_DIM), jnp.float32),
        pltpu.SemaphoreType.DMA((2,)),
        pltpu.SemaphoreType.DMA((2,)),
        pltpu.SemaphoreType.DMA((2,)),
        pltpu.SemaphoreType.DMA((2,)),
    ],
)
def _copy_rows(emb_hbm, out_hbm, stage_t, stage_s, in_t, out_t, in_s, out_s):
    c = lax.axis_index("c")
    s = lax.axis_index("s")
    base = (s * _NC + c) * _ROWS_PER_W

    def mk(stage, in_sems, out_sems, off):
        def in_copy(i):
            return pltpu.make_async_copy(
                emb_hbm.at[pl.ds(base + (off + i) * _CH, _CH)],
                stage.at[i % 2],
                in_sems.at[i % 2],
            )

        def out_copy(i):
            return pltpu.make_async_copy(
                stage.at[i % 2],
                out_hbm.at[pl.ds(base + (off + i) * _CH, _CH)],
                out_sems.at[i % 2],
            )

        return in_copy, out_copy

    split = 6
    a_in, a_out = mk(stage_t, in_t, out_t, 0)
    b_in, b_out = mk(stage_s.at[s], in_s, out_s, split)
    ops_a = _pipeline_ops(split)
    ops_b = _pipeline_ops(_NCHUNK - split)
    run = {
        "si": lambda f, i: f[0](i).start(),
        "so": lambda f, i: f[1](i).start(),
        "wi": lambda f, i: f[0](i).wait(),
        "wo": lambda f, i: f[1](i).wait(),
    }
    for j in range(max(len(ops_a), len(ops_b))):
        if j < len(ops_a):
            k, i = ops_a[j]
            run[k]((a_in, a_out), i)
        if j < len(ops_b):
            k, i = ops_b[j]
            run[k]((b_in, b_out), i)


def kernel(x, embedding):
    del x
    return _copy_rows(embedding)
